# baseline (speedup 1.0000x reference)
import jax
import jax.numpy as jnp
from jax.experimental import pallas as pl
from jax.experimental.pallas import tpu as pltpu

_BM = 128


def _probe_kernel(x_ref, embs_ref, out_ref):
    ones = jnp.zeros((16000, 16), jnp.float32) + embs_ref[0, 0]
    out_ref[...] = jnp.dot(x_ref[...], ones, preferred_element_type=jnp.float32)


def kernel(ids, embs):
    b, v = ids.shape
    _, d = embs.shape
    x2 = ids.reshape(b // 16, v * 16)
    return pl.pallas_call(
        _probe_kernel,
        grid=(x2.shape[0] // _BM,),
        in_specs=[
            pl.BlockSpec((_BM, x2.shape[1]), lambda i: (i, 0)),
            pl.BlockSpec((1000, 16), lambda i: (0, 0)),
        ],
        out_specs=pl.BlockSpec((_BM, d), lambda i: (i, 0)),
        out_shape=jax.ShapeDtypeStruct((x2.shape[0], d), jnp.float32),
        compiler_params=pltpu.CompilerParams(
            dimension_semantics=("arbitrary",)
        ),
    )(x2, embs)


# speedup vs baseline: 1.6880x; 1.6880x over previous
import jax
import jax.numpy as jnp
from jax.experimental import pallas as pl
from jax.experimental.pallas import tpu as pltpu


def _embed_kernel(ids_ref, embs_ref, out_ref):
    e = embs_ref[...]
    norm = jnp.sqrt(jnp.sum(e * e, axis=1, keepdims=True))
    normed = e / jnp.maximum(norm, 1e-12)
    out_ref[...] = jnp.dot(
        ids_ref[...], normed, preferred_element_type=jnp.float32
    )


def kernel(ids, embs):
    b, v = ids.shape
    _, d = embs.shape
    bm = 2048
    return pl.pallas_call(
        _embed_kernel,
        grid=(b // bm,),
        in_specs=[
            pl.BlockSpec((bm, v), lambda i: (i, 0)),
            pl.BlockSpec((v, d), lambda i: (0, 0)),
        ],
        out_specs=pl.BlockSpec((bm, d), lambda i: (i, 0)),
        out_shape=jax.ShapeDtypeStruct((b, d), jnp.float32),
        compiler_params=pltpu.CompilerParams(
            dimension_semantics=("arbitrary",)
        ),
    )(ids, embs)


# speedup vs baseline: 7.0929x; 4.2018x over previous
import jax
import jax.numpy as jnp
from jax.experimental import pallas as pl
from jax.experimental.pallas import tpu as pltpu

_BN = 2048


def _embed_kernel(embs_t_ref, ids_t_ref, out_ref):
    e = embs_t_ref[...]
    norm = jnp.sqrt(jnp.sum(e * e, axis=0, keepdims=True))
    normed = e / jnp.maximum(norm, 1e-12)
    out_ref[...] = jnp.dot(
        normed, ids_t_ref[...], preferred_element_type=jnp.float32
    )


def kernel(ids, embs):
    b, v = ids.shape
    _, d = embs.shape
    ids_t = ids.T
    embs_t = embs.T
    out_t = pl.pallas_call(
        _embed_kernel,
        grid=(b // _BN,),
        in_specs=[
            pl.BlockSpec((d, v), lambda i: (0, 0)),
            pl.BlockSpec((v, _BN), lambda i: (0, i)),
        ],
        out_specs=pl.BlockSpec((d, _BN), lambda i: (0, i)),
        out_shape=jax.ShapeDtypeStruct((d, b), jnp.float32),
        compiler_params=pltpu.CompilerParams(
            dimension_semantics=("arbitrary",)
        ),
    )(embs_t, ids_t)
    return out_t.T
